# baseline (speedup 1.0000x reference)
import functools
import jax
import jax.numpy as jnp
from jax.experimental import pallas as pl
from jax.experimental.pallas import tpu as pltpu

E = 8
TOP_K = 2
H = 768
FF = 4 * H
T = 2048

BT = 256
BF = 768
NT = T // BT
NF = FF // BF


def _router_body(hs_ref, gw_ref, gb_ref, noise_ref,
                 logits_ref, sel_ref, w_ref, coeff_ref):
    x = hs_ref[...]
    logits = jnp.dot(x, gw_ref[...], preferred_element_type=jnp.float32)
    logits = logits + gb_ref[...] + noise_ref[...]
    logits_ref[...] = logits
    m = jnp.max(logits, axis=-1, keepdims=True)
    p = jnp.exp(logits - m)
    probs = p / jnp.sum(p, axis=-1, keepdims=True)
    iota = jax.lax.broadcasted_iota(jnp.int32, (T, E), 1)
    m1 = jnp.max(probs, axis=-1, keepdims=True)
    i1 = jnp.min(jnp.where(probs == m1, iota, E), axis=-1, keepdims=True)
    probs2 = jnp.where(iota == i1, -1.0, probs)
    m2 = jnp.max(probs2, axis=-1, keepdims=True)
    i2 = jnp.min(jnp.where(probs2 == m2, iota, E), axis=-1, keepdims=True)
    s = m1 + m2
    w0 = m1 / s
    w1 = m2 / s
    sel_ref[...] = jnp.concatenate([i1, i2], axis=1)
    w_ref[...] = jnp.concatenate([w0, w1], axis=1)
    coeff_ref[...] = (jnp.where(iota == i1, w0, 0.0)
                      + jnp.where(iota == i2, w1, 0.0))


def _run_router(hs, gate_w, gate_b, noise):
    return pl.pallas_call(
        _router_body,
        out_shape=(
            jax.ShapeDtypeStruct((T, E), jnp.float32),
            jax.ShapeDtypeStruct((T, TOP_K), jnp.int32),
            jax.ShapeDtypeStruct((T, TOP_K), jnp.float32),
            jax.ShapeDtypeStruct((T, E), jnp.float32),
        ),
    )(hs, gate_w, gate_b.reshape(1, E), noise)


def _dense_body(coeff_ref, x_ref, w1_ref, b1_ref, w3_ref, b3_ref,
                w2_ref, b2_ref, out_ref, acc_ref):
    e = pl.program_id(1)
    f = pl.program_id(2)
    x = x_ref[...]
    h1 = jnp.dot(x, w1_ref[0], preferred_element_type=jnp.float32) + b1_ref[0]
    h1 = h1 / (1.0 + jnp.exp(-h1))
    h3 = jnp.dot(x, w3_ref[0], preferred_element_type=jnp.float32) + b3_ref[0]
    pp = jnp.dot(h1 * h3, w2_ref[0], preferred_element_type=jnp.float32)

    @pl.when(f == 0)
    def _():
        acc_ref[...] = pp

    @pl.when(f > 0)
    def _():
        acc_ref[...] += pp

    @pl.when(f == NF - 1)
    def _():
        y = acc_ref[...] + b2_ref[0]
        cb = coeff_ref[...]
        iota = jax.lax.broadcasted_iota(jnp.int32, (BT, E), 1)
        c = jnp.sum(jnp.where(iota == e, cb, 0.0), axis=-1, keepdims=True)
        contrib = y * c

        @pl.when(e == 0)
        def _():
            out_ref[...] = contrib

        @pl.when(e > 0)
        def _():
            out_ref[...] += contrib


def _run_dense(coeff, hs, w1, b1, w2, b2, w3, b3):
    return pl.pallas_call(
        _dense_body,
        grid=(NT, E, NF),
        in_specs=[
            pl.BlockSpec((BT, E), lambda t, e, f: (t, 0)),
            pl.BlockSpec((BT, H), lambda t, e, f: (t, 0)),
            pl.BlockSpec((1, H, BF), lambda t, e, f: (e, 0, f)),
            pl.BlockSpec((1, 1, BF), lambda t, e, f: (e, 0, f)),
            pl.BlockSpec((1, H, BF), lambda t, e, f: (e, 0, f)),
            pl.BlockSpec((1, 1, BF), lambda t, e, f: (e, 0, f)),
            pl.BlockSpec((1, BF, H), lambda t, e, f: (e, f, 0)),
            pl.BlockSpec((1, 1, H), lambda t, e, f: (e, 0, 0)),
        ],
        out_specs=pl.BlockSpec((BT, H), lambda t, e, f: (t, 0)),
        out_shape=jax.ShapeDtypeStruct((T, H), jnp.float32),
        scratch_shapes=[pltpu.VMEM((BT, H), jnp.float32)],
    )(coeff, hs, w1, b1.reshape(E, 1, FF), w3, b3.reshape(E, 1, FF),
      w2, b2.reshape(E, 1, H))


def kernel(x, gate_w, gate_b, w1, b1, w2, b2, w3, b3):
    Bd, Td, Hd = x.shape
    hs = x.reshape(-1, Hd)
    noise = jax.random.normal(jax.random.key(42), (T, E), jnp.float32) * 0.01
    logits, sel, _w, coeff = _run_router(hs, gate_w, gate_b, noise)
    out = _run_dense(coeff, hs, w1, b1, w2, b2, w3, b3)
    return (out.reshape(Bd, Td, Hd), logits, sel)


# speedup vs baseline: 1.5893x; 1.5893x over previous
import functools
import jax
import jax.numpy as jnp
from jax import lax
from jax.experimental import pallas as pl
from jax.experimental.pallas import tpu as pltpu
from jax.experimental.pallas import tpu_sc as plsc

E = 8
H = 768
FF = 4 * H
T = 2048
N = 2 * T
BM = 128
NB = N // BM
G = NB + E - 1
BF = 768
NF = FF // BF

NW = 32
TW = T // NW
CB = 128


def _router_body(hs_ref, gw_ref, gb_ref, noise_ref,
                 logits_ref, sel_ref, w_ref, p0_ref, p1_ref, cnt_ref):
    x = hs_ref[...]
    logits = jnp.dot(x, gw_ref[...], preferred_element_type=jnp.float32)
    logits = logits + gb_ref[...] + noise_ref[...]
    logits_ref[...] = logits
    m = jnp.max(logits, axis=-1, keepdims=True)
    p = jnp.exp(logits - m)
    probs = p / jnp.sum(p, axis=-1, keepdims=True)
    iota = lax.broadcasted_iota(jnp.int32, (T, E), 1)
    m1 = jnp.max(probs, axis=-1, keepdims=True)
    i1 = jnp.min(jnp.where(probs == m1, iota, E), axis=-1, keepdims=True)
    probs2 = jnp.where(iota == i1, -1.0, probs)
    m2 = jnp.max(probs2, axis=-1, keepdims=True)
    i2 = jnp.min(jnp.where(probs2 == m2, iota, E), axis=-1, keepdims=True)
    s = m1 + m2
    sel_ref[...] = jnp.concatenate([i1, i2], axis=1)
    w_ref[...] = jnp.concatenate([m1 / s, m2 / s], axis=1)

    oh1 = jnp.where(iota == i1, 1.0, 0.0)
    oh2 = jnp.where(iota == i2, 1.0, 0.0)
    S = oh1 + oh2
    r = lax.broadcasted_iota(jnp.int32, (CB, CB), 0)
    c = lax.broadcasted_iota(jnp.int32, (CB, CB), 1)
    L = jnp.where(r > c, 1.0, 0.0)
    blocks = []
    running = jnp.zeros((1, E), jnp.float32)
    for k in range(T // CB):
        Sk = S[k * CB:(k + 1) * CB, :]
        blocks.append(jnp.dot(L, Sk, preferred_element_type=jnp.float32)
                      + running)
        running = running + jnp.sum(Sk, axis=0, keepdims=True)
    csum = jnp.concatenate(blocks, axis=0)
    counts = running
    z1 = jnp.zeros((1, 1), jnp.float32)
    offs = jnp.concatenate([z1, counts[:, :-1]], axis=1)
    offs = offs + jnp.concatenate([z1, offs[:, :-1]], axis=1)
    z2 = jnp.zeros((1, 2), jnp.float32)
    offs = offs + jnp.concatenate([z2, offs[:, :-2]], axis=1)
    z4 = jnp.zeros((1, 4), jnp.float32)
    offs = offs + jnp.concatenate([z4, offs[:, :-4]], axis=1)
    base = csum + offs
    pos0 = jnp.sum(jnp.where(iota == i1, base, 0.0), axis=-1, keepdims=True)
    pos1 = jnp.sum(jnp.where(iota == i2, base, 0.0), axis=-1, keepdims=True)
    p0_ref[...] = pos0.astype(jnp.int32)
    p1_ref[...] = pos1.astype(jnp.int32)
    cnt_ref[...] = counts.astype(jnp.int32)


def _run_router(hs, gate_w, gate_b, noise):
    return pl.pallas_call(
        _router_body,
        out_shape=(
            jax.ShapeDtypeStruct((T, E), jnp.float32),
            jax.ShapeDtypeStruct((T, 2), jnp.int32),
            jax.ShapeDtypeStruct((T, 2), jnp.float32),
            jax.ShapeDtypeStruct((T, 1), jnp.int32),
            jax.ShapeDtypeStruct((T, 1), jnp.int32),
            jax.ShapeDtypeStruct((1, E), jnp.int32),
        ),
    )(hs, gate_w, gate_b.reshape(1, E), noise)


def _sc_dispatch(hs, p0, p1):
    mesh = plsc.VectorSubcoreMesh(core_axis_name="c", subcore_axis_name="s")

    @functools.partial(
        pl.kernel,
        out_type=jax.ShapeDtypeStruct((N, H), jnp.float32),
        mesh=mesh,
        scratch_types=[
            pltpu.VMEM((TW, H), jnp.float32),
            pltpu.VMEM((TW,), jnp.int32),
            pltpu.VMEM((TW,), jnp.int32),
            pltpu.SemaphoreType.DMA,
        ],
    )
    def k(hs_hbm, p0_hbm, p1_hbm, xs_hbm, xloc, p0v, p1v, sem):
        wid = lax.axis_index("s") * 2 + lax.axis_index("c")
        base = wid * TW
        pltpu.sync_copy(hs_hbm.at[pl.ds(base, TW)], xloc)
        pltpu.sync_copy(p0_hbm.at[pl.ds(base, TW)], p0v)
        pltpu.sync_copy(p1_hbm.at[pl.ds(base, TW)], p1v)
        pltpu.async_copy(xloc, xs_hbm.at[p0v], sem).wait()
        pltpu.async_copy(xloc, xs_hbm.at[p1v], sem).wait()

    return k(hs, p0, p1)


def _sc_gather(ys, p0, p1):
    mesh = plsc.VectorSubcoreMesh(core_axis_name="c", subcore_axis_name="s")

    @functools.partial(
        pl.kernel,
        out_type=(jax.ShapeDtypeStruct((T, H), jnp.float32),
                  jax.ShapeDtypeStruct((T, H), jnp.float32)),
        mesh=mesh,
        scratch_types=[
            pltpu.VMEM((TW, H), jnp.float32),
            pltpu.VMEM((TW, H), jnp.float32),
            pltpu.VMEM((TW,), jnp.int32),
            pltpu.VMEM((TW,), jnp.int32),
            pltpu.SemaphoreType.DMA,
        ],
    )
    def k(ys_hbm, p0_hbm, p1_hbm, y0_hbm, y1_hbm, y0loc, y1loc, p0v, p1v,
          sem):
        wid = lax.axis_index("s") * 2 + lax.axis_index("c")
        base = wid * TW
        pltpu.sync_copy(p0_hbm.at[pl.ds(base, TW)], p0v)
        pltpu.sync_copy(p1_hbm.at[pl.ds(base, TW)], p1v)
        pltpu.async_copy(ys_hbm.at[p0v], y0loc, sem).wait()
        pltpu.async_copy(ys_hbm.at[p1v], y1loc, sem).wait()
        pltpu.sync_copy(y0loc, y0_hbm.at[pl.ds(base, TW)])
        pltpu.sync_copy(y1loc, y1_hbm.at[pl.ds(base, TW)])

    return k(ys, p0, p1)


def _grouped_body(meta_ref, xs_ref, w1_ref, b1_ref, w3_ref, b3_ref,
                  w2_ref, b2_ref, out_ref, acc_ref):
    g = pl.program_id(0)
    f = pl.program_id(1)
    x = xs_ref[...]
    h1 = jnp.dot(x, w1_ref[0], preferred_element_type=jnp.float32) + b1_ref[0]
    h1 = h1 / (1.0 + jnp.exp(-h1))
    h3 = jnp.dot(x, w3_ref[0], preferred_element_type=jnp.float32) + b3_ref[0]
    pp = jnp.dot(h1 * h3, w2_ref[0], preferred_element_type=jnp.float32)

    @pl.when(f == 0)
    def _():
        acc_ref[...] = pp

    @pl.when(f > 0)
    def _():
        acc_ref[...] += pp

    @pl.when(f == NF - 1)
    def _():
        y = acc_ref[...] + b2_ref[0]
        lo = meta_ref[2, g]
        hi = meta_ref[3, g]
        b = meta_ref[1, g]
        rows = b * BM + lax.broadcasted_iota(jnp.int32, (BM, 1), 0)
        contrib = jnp.where((rows >= lo) & (rows < hi), y, 0.0)

        @pl.when(meta_ref[4, g] == 1)
        def _():
            out_ref[...] = contrib

        @pl.when(meta_ref[4, g] == 0)
        def _():
            out_ref[...] += contrib


def _run_grouped(meta, xs, w1, b1, w3, b3, w2, b2):
    grid_spec = pltpu.PrefetchScalarGridSpec(
        num_scalar_prefetch=1,
        grid=(G, NF),
        in_specs=[
            pl.BlockSpec((BM, H), lambda g, f, m: (m[1, g], 0)),
            pl.BlockSpec((1, H, BF), lambda g, f, m: (m[0, g], 0, f)),
            pl.BlockSpec((1, 1, BF), lambda g, f, m: (m[0, g], 0, f)),
            pl.BlockSpec((1, H, BF), lambda g, f, m: (m[0, g], 0, f)),
            pl.BlockSpec((1, 1, BF), lambda g, f, m: (m[0, g], 0, f)),
            pl.BlockSpec((1, BF, H), lambda g, f, m: (m[0, g], f, 0)),
            pl.BlockSpec((1, 1, H), lambda g, f, m: (m[0, g], 0, 0)),
        ],
        out_specs=pl.BlockSpec((BM, H), lambda g, f, m: (m[1, g], 0)),
        scratch_shapes=[pltpu.VMEM((BM, H), jnp.float32)],
    )
    return pl.pallas_call(
        _grouped_body,
        grid_spec=grid_spec,
        out_shape=jax.ShapeDtypeStruct((N, H), jnp.float32),
    )(meta, xs, w1, b1.reshape(E, 1, FF), w3, b3.reshape(E, 1, FF),
      w2, b2.reshape(E, 1, H))


def _make_meta(counts):
    counts = counts.reshape(E)
    offs = jnp.concatenate([jnp.zeros((1,), jnp.int32), jnp.cumsum(counts)])
    bstart = offs[:E] // BM
    nsteps = jnp.where(counts > 0, (offs[1:] + BM - 1) // BM - bstart, 0)
    sstart = jnp.concatenate([jnp.zeros((1,), jnp.int32),
                              jnp.cumsum(nsteps)])
    total = sstart[E]
    g = jnp.arange(G, dtype=jnp.int32)
    e_g = jnp.sum((g[:, None] >= sstart[None, 1:]).astype(jnp.int32), axis=1)
    valid = g < total
    e_c = jnp.clip(e_g, 0, E - 1)
    b_g = bstart[e_c] + (g - sstart[e_c])
    lo = jnp.maximum(offs[e_c], b_g * BM)
    hi = jnp.minimum(offs[e_c + 1], (b_g + 1) * BM)
    b_g = jnp.where(valid, b_g, NB - 1)
    lo = jnp.where(valid, lo, 0)
    hi = jnp.where(valid, hi, 0)
    e_f = jnp.where(valid, e_c, E - 1)
    fv = jnp.concatenate([jnp.ones((1,), jnp.int32),
                          (b_g[1:] != b_g[:-1]).astype(jnp.int32)])
    return jnp.stack([e_f, b_g, lo, hi, fv]).astype(jnp.int32)


def _combine_body(w_ref, y0_ref, y1_ref, out_ref):
    w = w_ref[...]
    out_ref[...] = (w[:, 0:1] * y0_ref[...] + w[:, 1:2] * y1_ref[...])


def _run_combine(w01, y0, y1):
    return pl.pallas_call(
        _combine_body,
        out_shape=jax.ShapeDtypeStruct((T, H), jnp.float32),
    )(w01, y0, y1)


def kernel(x, gate_w, gate_b, w1, b1, w2, b2, w3, b3):
    Bd, Td, Hd = x.shape
    hs = x.reshape(-1, Hd)
    noise = jax.random.normal(jax.random.key(42), (T, E), jnp.float32) * 0.01
    logits, sel, w01, p0, p1, counts = _run_router(hs, gate_w, gate_b, noise)
    p0 = p0.reshape(T)
    p1 = p1.reshape(T)
    xs = _sc_dispatch(hs, p0, p1)
    meta = _make_meta(counts)
    ys = _run_grouped(meta, xs, w1, b1, w3, b3, w2, b2)
    y0, y1 = _sc_gather(ys, p0, p1)
    out = _run_combine(w01, y0, y1)
    return (out.reshape(Bd, Td, Hd), logits, sel)


# speedup vs baseline: 3.0981x; 1.9494x over previous
import functools
import jax
import jax.numpy as jnp
from jax import lax
from jax.experimental import pallas as pl
from jax.experimental.pallas import tpu as pltpu
from jax.experimental.pallas import tpu_sc as plsc

E = 8
H = 768
FF = 4 * H
T = 2048
N = 2 * T
BM = 256
NB = N // BM
G = NB + E - 1
BF = 768
NF = FF // BF
G2 = NF * G

NW = 32
TW = T // NW
CB = 128


def _router_body(hs_ref, gw_ref, gb_ref, noise_ref,
                 logits_ref, sel_ref, w_ref, p0_ref, p1_ref, cnt_ref):
    x = hs_ref[...]
    logits = jnp.dot(x, gw_ref[...], preferred_element_type=jnp.float32)
    logits = logits + gb_ref[...] + noise_ref[...]
    logits_ref[...] = logits
    m = jnp.max(logits, axis=-1, keepdims=True)
    p = jnp.exp(logits - m)
    probs = p / jnp.sum(p, axis=-1, keepdims=True)
    iota = lax.broadcasted_iota(jnp.int32, (T, E), 1)
    m1 = jnp.max(probs, axis=-1, keepdims=True)
    i1 = jnp.min(jnp.where(probs == m1, iota, E), axis=-1, keepdims=True)
    probs2 = jnp.where(iota == i1, -1.0, probs)
    m2 = jnp.max(probs2, axis=-1, keepdims=True)
    i2 = jnp.min(jnp.where(probs2 == m2, iota, E), axis=-1, keepdims=True)
    s = m1 + m2
    sel_ref[...] = jnp.concatenate([i1, i2], axis=1)
    w_ref[...] = jnp.concatenate([m1 / s, m2 / s], axis=1)

    oh1 = jnp.where(iota == i1, 1.0, 0.0)
    oh2 = jnp.where(iota == i2, 1.0, 0.0)
    S = oh1 + oh2
    r = lax.broadcasted_iota(jnp.int32, (CB, CB), 0)
    c = lax.broadcasted_iota(jnp.int32, (CB, CB), 1)
    L = jnp.where(r > c, 1.0, 0.0)
    blocks = []
    running = jnp.zeros((1, E), jnp.float32)
    for k in range(T // CB):
        Sk = S[k * CB:(k + 1) * CB, :]
        blocks.append(jnp.dot(L, Sk, preferred_element_type=jnp.float32)
                      + running)
        running = running + jnp.sum(Sk, axis=0, keepdims=True)
    csum = jnp.concatenate(blocks, axis=0)
    counts = running
    z1 = jnp.zeros((1, 1), jnp.float32)
    offs = jnp.concatenate([z1, counts[:, :-1]], axis=1)
    offs = offs + jnp.concatenate([z1, offs[:, :-1]], axis=1)
    z2 = jnp.zeros((1, 2), jnp.float32)
    offs = offs + jnp.concatenate([z2, offs[:, :-2]], axis=1)
    z4 = jnp.zeros((1, 4), jnp.float32)
    offs = offs + jnp.concatenate([z4, offs[:, :-4]], axis=1)
    base = csum + offs
    pos0 = jnp.sum(jnp.where(iota == i1, base, 0.0), axis=-1, keepdims=True)
    pos1 = jnp.sum(jnp.where(iota == i2, base, 0.0), axis=-1, keepdims=True)
    p0_ref[...] = pos0.astype(jnp.int32)
    p1_ref[...] = pos1.astype(jnp.int32)
    total = offs[:, -1:] + counts[:, -1:]
    z7 = jnp.zeros((1, 7), jnp.float32)
    cnt_ref[...] = jnp.concatenate([offs, total, z7],
                                   axis=1).astype(jnp.int32)


def _run_router(hs, gate_w, gate_b, noise):
    return pl.pallas_call(
        _router_body,
        out_shape=(
            jax.ShapeDtypeStruct((T, E), jnp.float32),
            jax.ShapeDtypeStruct((T, 2), jnp.int32),
            jax.ShapeDtypeStruct((T, 2), jnp.float32),
            jax.ShapeDtypeStruct((T, 1), jnp.int32),
            jax.ShapeDtypeStruct((T, 1), jnp.int32),
            jax.ShapeDtypeStruct((1, 16), jnp.int32),
        ),
    )(hs, gate_w, gate_b.reshape(1, E), noise)


def _sc_dispatch(hs, p0, p1):
    mesh = plsc.VectorSubcoreMesh(core_axis_name="c", subcore_axis_name="s")

    @functools.partial(
        pl.kernel,
        out_type=jax.ShapeDtypeStruct((N, H), jnp.float32),
        mesh=mesh,
        scratch_types=[
            pltpu.VMEM((TW, H), jnp.float32),
            pltpu.VMEM((TW,), jnp.int32),
            pltpu.VMEM((TW,), jnp.int32),
            pltpu.SemaphoreType.DMA,
        ],
    )
    def k(hs_hbm, p0_hbm, p1_hbm, xs_hbm, xloc, p0v, p1v, sem):
        wid = lax.axis_index("s") * 2 + lax.axis_index("c")
        base = wid * TW
        pltpu.sync_copy(hs_hbm.at[pl.ds(base, TW)], xloc)
        pltpu.sync_copy(p0_hbm.at[pl.ds(base, TW)], p0v)
        pltpu.sync_copy(p1_hbm.at[pl.ds(base, TW)], p1v)
        pltpu.async_copy(xloc, xs_hbm.at[p0v], sem).wait()
        pltpu.async_copy(xloc, xs_hbm.at[p1v], sem).wait()

    return k(hs, p0, p1)


def _sc_gather(ys, p0, p1):
    mesh = plsc.VectorSubcoreMesh(core_axis_name="c", subcore_axis_name="s")

    @functools.partial(
        pl.kernel,
        out_type=(jax.ShapeDtypeStruct((T, H), jnp.float32),
                  jax.ShapeDtypeStruct((T, H), jnp.float32)),
        mesh=mesh,
        scratch_types=[
            pltpu.VMEM((TW, H), jnp.float32),
            pltpu.VMEM((TW, H), jnp.float32),
            pltpu.VMEM((TW,), jnp.int32),
            pltpu.VMEM((TW,), jnp.int32),
            pltpu.SemaphoreType.DMA,
        ],
    )
    def k(ys_hbm, p0_hbm, p1_hbm, y0_hbm, y1_hbm, y0loc, y1loc, p0v, p1v,
          sem):
        wid = lax.axis_index("s") * 2 + lax.axis_index("c")
        base = wid * TW
        pltpu.sync_copy(p0_hbm.at[pl.ds(base, TW)], p0v)
        pltpu.sync_copy(p1_hbm.at[pl.ds(base, TW)], p1v)
        pltpu.async_copy(ys_hbm.at[p0v], y0loc, sem).wait()
        pltpu.async_copy(ys_hbm.at[p1v], y1loc, sem).wait()
        pltpu.sync_copy(y0loc, y0_hbm.at[pl.ds(base, TW)])
        pltpu.sync_copy(y1loc, y1_hbm.at[pl.ds(base, TW)])

    return k(ys, p0, p1)


def _grouped_body(offs_ref, xs_ref, w1_ref, b1_ref, w3_ref, b3_ref,
                  w2_ref, b2_ref, out_ref):
    e = pl.program_id(0)
    f = pl.program_id(1)
    start = offs_ref[0, e]
    end = offs_ref[0, e + 1]
    blo = start // BM
    bhi = jnp.where(end > start, (end + BM - 1) // BM, blo)
    def step(b, carry):
        sl = pl.ds(b * BM, BM)
        x = xs_ref[sl, :]
        h1 = (jnp.dot(x, w1_ref[0], preferred_element_type=jnp.float32)
              + b1_ref[0])
        h1 = h1 / (1.0 + jnp.exp(-h1))
        h3 = (jnp.dot(x, w3_ref[0], preferred_element_type=jnp.float32)
              + b3_ref[0])
        pp = jnp.dot(h1 * h3, w2_ref[0], preferred_element_type=jnp.float32)
        rows = b * BM + lax.broadcasted_iota(jnp.int32, (BM, 1), 0)
        mask = (rows >= start) & (rows < end)
        v = out_ref[sl, :]
        acc = jnp.where(f == 0, pp, v + pp)
        acc = jnp.where(f == NF - 1, acc + b2_ref[0], acc)
        out_ref[sl, :] = jnp.where(mask, acc, v)
        return carry

    lax.fori_loop(blo, bhi, step, 0)


def _run_grouped(offs, xs, w1, b1, w3, b3, w2, b2):
    grid_spec = pltpu.PrefetchScalarGridSpec(
        num_scalar_prefetch=1,
        grid=(E, NF),
        in_specs=[
            pl.BlockSpec((N, H), lambda e, f, o: (0, 0)),
            pl.BlockSpec((1, H, BF), lambda e, f, o: (e, 0, f)),
            pl.BlockSpec((1, 1, BF), lambda e, f, o: (e, 0, f)),
            pl.BlockSpec((1, H, BF), lambda e, f, o: (e, 0, f)),
            pl.BlockSpec((1, 1, BF), lambda e, f, o: (e, 0, f)),
            pl.BlockSpec((1, BF, H), lambda e, f, o: (e, f, 0)),
            pl.BlockSpec((1, 1, H), lambda e, f, o: (e, 0, 0)),
        ],
        out_specs=pl.BlockSpec((N, H), lambda e, f, o: (0, 0)),
    )
    return pl.pallas_call(
        _grouped_body,
        grid_spec=grid_spec,
        out_shape=jax.ShapeDtypeStruct((N, H), jnp.float32),
    )(offs, xs, w1, b1.reshape(E, 1, FF), w3, b3.reshape(E, 1, FF),
      w2, b2.reshape(E, 1, H))


def _combine_body(w_ref, y0_ref, y1_ref, out_ref):
    w = w_ref[...]
    out_ref[...] = (w[:, 0:1] * y0_ref[...] + w[:, 1:2] * y1_ref[...])


def _run_combine(w01, y0, y1):
    return pl.pallas_call(
        _combine_body,
        out_shape=jax.ShapeDtypeStruct((T, H), jnp.float32),
    )(w01, y0, y1)


def kernel(x, gate_w, gate_b, w1, b1, w2, b2, w3, b3):
    Bd, Td, Hd = x.shape
    hs = x.reshape(-1, Hd)
    noise = jax.random.normal(jax.random.key(42), (T, E), jnp.float32) * 0.01
    logits, sel, w01, p0, p1, offs = _run_router(hs, gate_w, gate_b, noise)
    p0 = p0.reshape(T)
    p1 = p1.reshape(T)
    xs = _sc_dispatch(hs, p0, p1)
    ys = _run_grouped(offs, xs, w1, b1, w3, b3, w2, b2)
    y0, y1 = _sc_gather(ys, p0, p1)
    out = _run_combine(w01, y0, y1)
    return (out.reshape(Bd, Td, Hd), logits, sel)
